# Initial kernel scaffold; baseline (speedup 1.0000x reference)
import jax
import jax.numpy as jnp
from jax.experimental import pallas as pl


def kernel(values, row_indices, table):
    raise NotImplementedError("write your pallas kernel here")


# speedup vs baseline: 3.8613x; 3.8613x over previous
import functools

import jax
import jax.numpy as jnp
from jax import lax
from jax.experimental import pallas as pl
from jax.experimental.pallas import tpu as pltpu
from jax.experimental.pallas import tpu_sc as plsc

_BATCH = 1024
_SLOT = 26
_DIM = 32
_NSEG = _BATCH * _SLOT
_NNZ = _NSEG * 20
_NC = 2
_NS = 16
_NW = _NC * _NS
_P = _NNZ // _NW
_UNIT = 128
_NUNIT = _P // _UNIT
_SEG_T = _NSEG // _NS


def _sc_body(values_hbm, segids_hbm, table_hbm, acc_hbm, cnt_hbm,
             acc_s, cnt_s, idx_all, seg_all, seg_v, ones_v, rows_v, zcnt,
             gsem):
    c = lax.axis_index("c")
    s = lax.axis_index("s")
    base = (c * _NS + s) * _P

    zeros16 = jnp.zeros((16,), jnp.float32)
    ones16 = jnp.ones((16,), jnp.float32)

    def _zrow(r, carry):
        rows_v[r, pl.ds(0, 16)] = zeros16
        rows_v[r, pl.ds(16, 16)] = zeros16
        return carry
    lax.fori_loop(0, _UNIT, _zrow, 0)

    def _zc(i, carry):
        zcnt[pl.ds(i * 16, 16)] = zeros16
        return carry
    lax.fori_loop(0, _SEG_T // 16, _zc, 0)

    for j in range(_UNIT // 16):
        ones_v[pl.ds(j * 16, 16)] = ones16

    def _zacc(k, carry):
        pltpu.sync_copy(rows_v,
                        acc_s.at[pl.ds(s * _SEG_T + k * _UNIT, _UNIT)])
        return carry
    lax.fori_loop(0, _SEG_T // _UNIT, _zacc, 0)
    pltpu.sync_copy(zcnt, cnt_s.at[pl.ds(s * _SEG_T, _SEG_T)])

    pltpu.sync_copy(values_hbm.at[pl.ds(base, _P)], idx_all)
    pltpu.sync_copy(segids_hbm.at[pl.ds(base, _P)], seg_all)

    plsc.subcore_barrier()

    def _unit(u, carry):
        off = u * _UNIT
        for j in range(_UNIT // 16):
            seg_v[pl.ds(j * 16, 16)] = seg_all[pl.ds(off + j * 16, 16)]
        pltpu.async_copy(table_hbm.at[idx_all.at[pl.ds(off, _UNIT)]],
                         rows_v, gsem).wait()
        pltpu.sync_copy(rows_v, acc_s.at[seg_v], add=True)
        pltpu.sync_copy(ones_v, cnt_s.at[seg_v], add=True)
        return carry
    lax.fori_loop(0, _NUNIT, _unit, 0)

    plsc.subcore_barrier()

    row0 = s * _SEG_T
    pltpu.sync_copy(acc_s.at[pl.ds(row0, _SEG_T)],
                    acc_hbm.at[c, pl.ds(row0, _SEG_T)])
    pltpu.sync_copy(cnt_s.at[pl.ds(row0, _SEG_T)],
                    cnt_hbm.at[c, pl.ds(row0, _SEG_T)])


_sc_lookup = functools.partial(
    pl.kernel,
    out_type=(jax.ShapeDtypeStruct((_NC, _NSEG, _DIM), jnp.float32),
              jax.ShapeDtypeStruct((_NC, _NSEG), jnp.float32)),
    mesh=plsc.VectorSubcoreMesh(core_axis_name="c", subcore_axis_name="s",
                                num_cores=_NC, num_subcores=_NS),
    scratch_types=[
        pltpu.VMEM_SHARED((_NSEG, _DIM), jnp.float32),
        pltpu.VMEM_SHARED((_NSEG,), jnp.float32),
        pltpu.VMEM((_P,), jnp.int32),
        pltpu.VMEM((_P,), jnp.int32),
        pltpu.VMEM((_UNIT,), jnp.int32),
        pltpu.VMEM((_UNIT,), jnp.float32),
        pltpu.VMEM((_UNIT, _DIM), jnp.float32),
        pltpu.VMEM((_SEG_T,), jnp.float32),
        pltpu.SemaphoreType.DMA,
    ],
    compiler_params=pltpu.CompilerParams(use_tc_tiling_on_sc=False),
)(_sc_body)


def _combine_body(acc_ref, cnt_ref, out_ref):
    total = acc_ref[0] + acc_ref[1]
    cnt = jnp.maximum(cnt_ref[0] + cnt_ref[1], 1.0)
    out_ref[...] = total / cnt


_ROWS_BLK = 2048


def _combine(acc, cnt):
    grid = _NSEG // _ROWS_BLK
    return pl.pallas_call(
        _combine_body,
        grid=(grid,),
        in_specs=[
            pl.BlockSpec((_NC, _ROWS_BLK, _DIM), lambda i: (0, i, 0)),
            pl.BlockSpec((_NC, _ROWS_BLK, 1), lambda i: (0, i, 0)),
        ],
        out_specs=pl.BlockSpec((_ROWS_BLK, _DIM), lambda i: (i, 0)),
        out_shape=jax.ShapeDtypeStruct((_NSEG, _DIM), jnp.float32),
    )(acc, cnt.reshape(_NC, _NSEG, 1))


def kernel(values, row_indices, table):
    acc, cnt = _sc_lookup(values, row_indices, table)
    out = _combine(acc, cnt)
    return out.reshape(_BATCH, _SLOT, _DIM)


# speedup vs baseline: 4.2452x; 1.0994x over previous
import functools

import jax
import jax.numpy as jnp
from jax import lax
from jax.experimental import pallas as pl
from jax.experimental.pallas import tpu as pltpu
from jax.experimental.pallas import tpu_sc as plsc

_BATCH = 1024
_SLOT = 26
_DIM = 32
_NSEG = _BATCH * _SLOT
_NNZ = _NSEG * 20
_NC = 2
_NS = 16
_NW = _NC * _NS
_P = _NNZ // _NW
_UNIT = 832
_NUNIT = _P // _UNIT
_SEG_T = _NSEG // _NS


def _sc_body(values_hbm, segids_hbm, table_hbm, acc_hbm, cnt_hbm,
             acc_s, cnt_s, idx_all, seg_all, seg_v, ones_v, rows_v, zcnt,
             gsem):
    c = lax.axis_index("c")
    s = lax.axis_index("s")
    base = (c * _NS + s) * _P

    zeros16 = jnp.zeros((16,), jnp.float32)
    ones16 = jnp.ones((16,), jnp.float32)

    def _zrow(r, carry):
        rows_v[r, pl.ds(0, 16)] = zeros16
        rows_v[r, pl.ds(16, 16)] = zeros16
        return carry
    lax.fori_loop(0, _UNIT, _zrow, 0)

    def _zc(i, carry):
        zcnt[pl.ds(i * 16, 16)] = zeros16
        return carry
    lax.fori_loop(0, _SEG_T // 16, _zc, 0)

    for j in range(_UNIT // 16):
        ones_v[pl.ds(j * 16, 16)] = ones16

    def _zacc(k, carry):
        pltpu.sync_copy(rows_v,
                        acc_s.at[pl.ds(s * _SEG_T + k * _UNIT, _UNIT)])
        return carry
    lax.fori_loop(0, _SEG_T // _UNIT, _zacc, 0)
    pltpu.sync_copy(zcnt, cnt_s.at[pl.ds(s * _SEG_T, _SEG_T)])

    pltpu.sync_copy(values_hbm.at[pl.ds(base, _P)], idx_all)
    pltpu.sync_copy(segids_hbm.at[pl.ds(base, _P)], seg_all)

    plsc.subcore_barrier()

    def _unit(u, carry):
        off = u * _UNIT
        for j in range(_UNIT // 16):
            seg_v[pl.ds(j * 16, 16)] = seg_all[pl.ds(off + j * 16, 16)]
        pltpu.async_copy(table_hbm.at[idx_all.at[pl.ds(off, _UNIT)]],
                         rows_v, gsem).wait()
        pltpu.sync_copy(rows_v, acc_s.at[seg_v], add=True)
        pltpu.sync_copy(ones_v, cnt_s.at[seg_v], add=True)
        return carry
    lax.fori_loop(0, _NUNIT, _unit, 0)

    plsc.subcore_barrier()

    row0 = s * _SEG_T
    pltpu.sync_copy(acc_s.at[pl.ds(row0, _SEG_T)],
                    acc_hbm.at[c, pl.ds(row0, _SEG_T)])
    pltpu.sync_copy(cnt_s.at[pl.ds(row0, _SEG_T)],
                    cnt_hbm.at[c, pl.ds(row0, _SEG_T)])


_sc_lookup = functools.partial(
    pl.kernel,
    out_type=(jax.ShapeDtypeStruct((_NC, _NSEG, _DIM), jnp.float32),
              jax.ShapeDtypeStruct((_NC, _NSEG), jnp.float32)),
    mesh=plsc.VectorSubcoreMesh(core_axis_name="c", subcore_axis_name="s",
                                num_cores=_NC, num_subcores=_NS),
    scratch_types=[
        pltpu.VMEM_SHARED((_NSEG, _DIM), jnp.float32),
        pltpu.VMEM_SHARED((_NSEG,), jnp.float32),
        pltpu.VMEM((_P,), jnp.int32),
        pltpu.VMEM((_P,), jnp.int32),
        pltpu.VMEM((_UNIT,), jnp.int32),
        pltpu.VMEM((_UNIT,), jnp.float32),
        pltpu.VMEM((_UNIT, _DIM), jnp.float32),
        pltpu.VMEM((_SEG_T,), jnp.float32),
        pltpu.SemaphoreType.DMA,
    ],
    compiler_params=pltpu.CompilerParams(use_tc_tiling_on_sc=False),
)(_sc_body)


def _combine_body(acc_ref, cnt_ref, out_ref):
    total = acc_ref[0] + acc_ref[1]
    cnt = jnp.maximum(cnt_ref[0] + cnt_ref[1], 1.0)
    out_ref[...] = total / cnt


_ROWS_BLK = 2048


def _combine(acc, cnt):
    grid = _NSEG // _ROWS_BLK
    return pl.pallas_call(
        _combine_body,
        grid=(grid,),
        in_specs=[
            pl.BlockSpec((_NC, _ROWS_BLK, _DIM), lambda i: (0, i, 0)),
            pl.BlockSpec((_NC, _ROWS_BLK, 1), lambda i: (0, i, 0)),
        ],
        out_specs=pl.BlockSpec((_ROWS_BLK, _DIM), lambda i: (i, 0)),
        out_shape=jax.ShapeDtypeStruct((_NSEG, _DIM), jnp.float32),
    )(acc, cnt.reshape(_NC, _NSEG, 1))


def kernel(values, row_indices, table):
    acc, cnt = _sc_lookup(values, row_indices, table)
    out = _combine(acc, cnt)
    return out.reshape(_BATCH, _SLOT, _DIM)
